# baseline (speedup 1.0000x reference)
import jax
import jax.numpy as jnp
from jax import lax
from jax.experimental import pallas as pl
from jax.experimental.pallas import tpu as pltpu
from jax.experimental.pallas import tpu_sc as plsc

_N_MOL = 16
_N_REP = 4
_N_ATOM = 512
_L = 16
_HALF = _N_ATOM // 2
_T = _N_MOL * _N_REP * _N_ATOM
_MTOK = _N_REP * _N_ATOM


def _sc_body(p_hbm, e_hbm, h_hbm, fc_hbm, out_hbm, buf_v, fc_v, frac_v, out_v,
             sem_in, sem_fc):
    wid = lax.axis_index("s") * 2 + lax.axis_index("c")
    m = wid // 2
    half = wid % 2
    a0 = half * _HALF

    tok0 = m * _MTOK
    cp_p = pltpu.async_copy(
        p_hbm.at[pl.ds(tok0, _MTOK)], buf_v.at[pl.ds(0, _MTOK)], sem_in)
    cp_e = pltpu.async_copy(
        e_hbm.at[pl.ds(tok0, _MTOK)], buf_v.at[pl.ds(_MTOK, _MTOK)], sem_in)
    cp_h = pltpu.async_copy(
        h_hbm.at[pl.ds(tok0, _MTOK)], buf_v.at[pl.ds(2 * _MTOK, _MTOK)], sem_in)
    cp_fc = pltpu.async_copy(
        fc_hbm.at[pl.ds(tok0, _MTOK)], fc_v, sem_fc)
    cp_p.wait()
    cp_e.wait()
    cp_h.wait()
    cp_fc.wait()

    z = jnp.zeros((_L,), jnp.float32)

    def rep_body(r, carry):
        def grp_body(g, c):
            s_num, s_den = c
            t = r * _N_ATOM + g * _L
            p = buf_v[pl.ds(t, _L)]
            e = buf_v[pl.ds(_MTOK + t, _L)]
            h = buf_v[pl.ds(2 * _MTOK + t, _L)]
            fc = fc_v[pl.ds(t, _L)]
            invh = 1.0 / h
            u = p - e * invh
            return (s_num + (u - fc), s_den + invh)

        s_num, s_den = lax.fori_loop(0, _N_ATOM // _L, grp_body, (z, z),
                                     unroll=4)
        num = jnp.broadcast_to(jnp.sum(s_num), (_L,))
        den = jnp.broadcast_to(jnp.sum(s_den), (_L,))
        frac_v[pl.ds(r * _L, _L)] = num / den
        return carry

    lax.fori_loop(0, _N_REP, rep_body, 0)

    def out_body(g, carry):
        def rep2_body(r, acc):
            t = r * _N_ATOM + a0 + g * _L
            p = buf_v[pl.ds(t, _L)]
            e = buf_v[pl.ds(_MTOK + t, _L)]
            h = buf_v[pl.ds(2 * _MTOK + t, _L)]
            frac = frac_v[pl.ds(r * _L, _L)]
            return acc + (p - (e + frac) / h)

        acc = lax.fori_loop(0, _N_REP, rep2_body, z)
        out_v[pl.ds(g * _L, _L)] = acc * (1.0 / _N_REP)
        return carry

    lax.fori_loop(0, _HALF // _L, out_body, 0, unroll=2)
    pltpu.sync_copy(out_v, out_hbm.at[pl.ds(m * _N_ATOM + a0, _HALF)])


def kernel(inputs, formal_charge, n_atoms, n_representations, n_molecules):
    mesh = plsc.VectorSubcoreMesh(core_axis_name="c", subcore_axis_name="s")
    run = pl.kernel(
        _sc_body,
        out_type=jax.ShapeDtypeStruct((_N_MOL * _N_ATOM,), jnp.float32),
        mesh=mesh,
        compiler_params=pltpu.CompilerParams(needs_layout_passes=False),
        scratch_types=[
            pltpu.VMEM((3 * _MTOK,), jnp.float32),
            pltpu.VMEM((_MTOK,), jnp.float32),
            pltpu.VMEM((_N_REP * _L,), jnp.float32),
            pltpu.VMEM((_HALF,), jnp.float32),
            pltpu.SemaphoreType.DMA,
            pltpu.SemaphoreType.DMA,
        ],
    )
    out = run(inputs[:, 0], inputs[:, 1], inputs[:, 2], formal_charge)
    return out.reshape(-1, 1)


# speedup vs baseline: 1.0065x; 1.0065x over previous
import jax
import jax.numpy as jnp
from jax import lax
from jax.experimental import pallas as pl
from jax.experimental.pallas import tpu as pltpu
from jax.experimental.pallas import tpu_sc as plsc

_N_MOL = 16
_N_REP = 4
_N_ATOM = 512
_L = 16
_HALF = _N_ATOM // 2
_T = _N_MOL * _N_REP * _N_ATOM
_MTOK = _N_REP * _N_ATOM


def _sc_body(p_hbm, e_hbm, h_hbm, fc_hbm, out_hbm, buf_v, fc_v, frac_v, out_v,
             sem_in, sem_fc):
    wid = lax.axis_index("s") * 2 + lax.axis_index("c")
    m = wid // 2
    half = wid % 2
    a0 = half * _HALF

    tok0 = m * _MTOK
    first = [
        pltpu.async_copy(src.at[pl.ds(tok0, _N_ATOM)],
                         buf_v.at[pl.ds(i * _MTOK, _N_ATOM)], sem_in)
        for i, src in enumerate((p_hbm, e_hbm, h_hbm))
    ]
    cp_fc = pltpu.async_copy(fc_hbm.at[pl.ds(tok0, _MTOK)], fc_v, sem_fc)
    rest = [
        pltpu.async_copy(src.at[pl.ds(tok0 + _N_ATOM, _MTOK - _N_ATOM)],
                         buf_v.at[pl.ds(i * _MTOK + _N_ATOM, _MTOK - _N_ATOM)],
                         sem_in)
        for i, src in enumerate((p_hbm, e_hbm, h_hbm))
    ]

    z = jnp.zeros((_L,), jnp.float32)

    def make_grp_body(r):
        def grp_body(g, c):
            s_num, s_den = c
            t = r * _N_ATOM + g * _L
            p = buf_v[pl.ds(t, _L)]
            e = buf_v[pl.ds(_MTOK + t, _L)]
            h = buf_v[pl.ds(2 * _MTOK + t, _L)]
            fc = fc_v[pl.ds(t, _L)]
            invh = 1.0 / h
            u = p - e * invh
            return (s_num + (u - fc), s_den + invh)

        return grp_body

    def rep_sums(r, carry):
        s_num, s_den = lax.fori_loop(0, _N_ATOM // _L, make_grp_body(r),
                                     (z, z), unroll=2)
        num = jnp.broadcast_to(jnp.sum(s_num), (_L,))
        den = jnp.broadcast_to(jnp.sum(s_den), (_L,))
        frac_v[pl.ds(r * _L, _L)] = num / den
        return carry

    for c in first:
        c.wait()
    cp_fc.wait()
    rep_sums(0, 0)
    for c in rest:
        c.wait()
    lax.fori_loop(1, _N_REP, rep_sums, 0)

    def out_body(g, carry):
        def rep2_body(r, acc):
            t = r * _N_ATOM + a0 + g * _L
            p = buf_v[pl.ds(t, _L)]
            e = buf_v[pl.ds(_MTOK + t, _L)]
            h = buf_v[pl.ds(2 * _MTOK + t, _L)]
            frac = frac_v[pl.ds(r * _L, _L)]
            return acc + (p - (e + frac) / h)

        acc = lax.fori_loop(0, _N_REP, rep2_body, z)
        out_v[pl.ds(g * _L, _L)] = acc * (1.0 / _N_REP)
        return carry

    lax.fori_loop(0, _HALF // _L, out_body, 0)
    pltpu.sync_copy(out_v, out_hbm.at[pl.ds(m * _N_ATOM + a0, _HALF)])


def kernel(inputs, formal_charge, n_atoms, n_representations, n_molecules):
    mesh = plsc.VectorSubcoreMesh(core_axis_name="c", subcore_axis_name="s")
    run = pl.kernel(
        _sc_body,
        out_type=jax.ShapeDtypeStruct((_N_MOL * _N_ATOM,), jnp.float32),
        mesh=mesh,
        compiler_params=pltpu.CompilerParams(needs_layout_passes=False),
        scratch_types=[
            pltpu.VMEM((3 * _MTOK,), jnp.float32),
            pltpu.VMEM((_MTOK,), jnp.float32),
            pltpu.VMEM((_N_REP * _L,), jnp.float32),
            pltpu.VMEM((_HALF,), jnp.float32),
            pltpu.SemaphoreType.DMA,
            pltpu.SemaphoreType.DMA,
        ],
    )
    out = run(inputs[:, 0], inputs[:, 1], inputs[:, 2], formal_charge)
    return out.reshape(-1, 1)


# speedup vs baseline: 1.0136x; 1.0071x over previous
import jax
import jax.numpy as jnp
from jax import lax
from jax.experimental import pallas as pl
from jax.experimental.pallas import tpu as pltpu
from jax.experimental.pallas import tpu_sc as plsc

_N_MOL = 16
_N_REP = 4
_N_ATOM = 512
_L = 16
_HALF = _N_ATOM // 2
_T = _N_MOL * _N_REP * _N_ATOM
_MTOK = _N_REP * _N_ATOM


def _sc_body(p_hbm, e_hbm, h_hbm, fc_hbm, out_hbm, buf_v, fc_v, frac_v, out_v,
             sem_in, sem_fc):
    wid = lax.axis_index("s") * 2 + lax.axis_index("c")
    m = wid // 2
    half = wid % 2
    a0 = half * _HALF

    tok0 = m * _MTOK
    cp_p = pltpu.async_copy(
        p_hbm.at[pl.ds(tok0, _MTOK)], buf_v.at[pl.ds(0, _MTOK)], sem_in)
    cp_e = pltpu.async_copy(
        e_hbm.at[pl.ds(tok0, _MTOK)], buf_v.at[pl.ds(_MTOK, _MTOK)], sem_in)
    cp_h = pltpu.async_copy(
        h_hbm.at[pl.ds(tok0, _MTOK)], buf_v.at[pl.ds(2 * _MTOK, _MTOK)], sem_in)
    cp_fc = pltpu.async_copy(
        fc_hbm.at[pl.ds(tok0, _MTOK)], fc_v, sem_fc)
    cp_p.wait()
    cp_e.wait()
    cp_h.wait()
    cp_fc.wait()

    z = jnp.zeros((_L,), jnp.float32)

    def rep_body(r, carry):
        def grp_body(g, c):
            s_num, s_den = c
            t = r * _N_ATOM + g * _L
            p = buf_v[pl.ds(t, _L)]
            e = buf_v[pl.ds(_MTOK + t, _L)]
            h = buf_v[pl.ds(2 * _MTOK + t, _L)]
            fc = fc_v[pl.ds(t, _L)]
            invh = 1.0 / h
            u = p - e * invh
            return (s_num + (u - fc), s_den + invh)

        s_num, s_den = lax.fori_loop(0, _N_ATOM // _L, grp_body, (z, z),
                                     unroll=2)
        num = jnp.broadcast_to(jnp.sum(s_num), (_L,))
        den = jnp.broadcast_to(jnp.sum(s_den), (_L,))
        frac_v[pl.ds(r * _L, _L)] = num / den
        return carry

    lax.fori_loop(0, _N_REP, rep_body, 0)

    def out_body(g, carry):
        def rep2_body(r, acc):
            t = r * _N_ATOM + a0 + g * _L
            p = buf_v[pl.ds(t, _L)]
            e = buf_v[pl.ds(_MTOK + t, _L)]
            h = buf_v[pl.ds(2 * _MTOK + t, _L)]
            frac = frac_v[pl.ds(r * _L, _L)]
            return acc + (p - (e + frac) / h)

        acc = lax.fori_loop(0, _N_REP, rep2_body, z)
        out_v[pl.ds(g * _L, _L)] = acc * (1.0 / _N_REP)
        return carry

    lax.fori_loop(0, _HALF // _L, out_body, 0)
    pltpu.sync_copy(out_v, out_hbm.at[pl.ds(m * _N_ATOM + a0, _HALF)])


def kernel(inputs, formal_charge, n_atoms, n_representations, n_molecules):
    mesh = plsc.VectorSubcoreMesh(core_axis_name="c", subcore_axis_name="s")
    run = pl.kernel(
        _sc_body,
        out_type=jax.ShapeDtypeStruct((_N_MOL * _N_ATOM,), jnp.float32),
        mesh=mesh,
        compiler_params=pltpu.CompilerParams(needs_layout_passes=False),
        scratch_types=[
            pltpu.VMEM((3 * _MTOK,), jnp.float32),
            pltpu.VMEM((_MTOK,), jnp.float32),
            pltpu.VMEM((_N_REP * _L,), jnp.float32),
            pltpu.VMEM((_HALF,), jnp.float32),
            pltpu.SemaphoreType.DMA,
            pltpu.SemaphoreType.DMA,
        ],
    )
    out = run(inputs[:, 0], inputs[:, 1], inputs[:, 2], formal_charge)
    return out.reshape(-1, 1)
